# baseline (speedup 1.0000x reference)
import functools

import jax
import jax.numpy as jnp
from jax import lax
from jax.experimental import pallas as pl
from jax.experimental.pallas import tpu as pltpu
from jax.experimental.pallas import tpu_sc as plsc

B = 16384
IN_DIM = 52
N_EMB = 26
VOCAB = 100000
EMB = 16
OUT_DIM = N_EMB * EMB + N_EMB

NC = 2
NS = 16
L = 16
NW = NC * NS
RPW = B // NW
NCHUNK = RPW // L
IDXW = 128
NIDX = RPW // IDXW


def _body(x_hbm, tab_hbm, out_hbm, x_v, idx_v, g_v, f_v, sem):
    wid = lax.axis_index("s") * NC + lax.axis_index("c")
    base = wid * RPW
    pltpu.sync_copy(x_hbm.at[pl.ds(base * IN_DIM, RPW * IN_DIM)], x_v)

    iota = lax.iota(jnp.int32, L)

    def col_body(i, carry):
        def idx_chunk(k, c):
            flat = (k * L + iota) * IN_DIM + i
            vals = plsc.load_gather(x_v, [flat]) + i * VOCAB
            idx_v[pl.ds(k * L, L)] = vals
            return c
        lax.fori_loop(0, NCHUNK, idx_chunk, 0)
        cps = [
            pltpu.async_copy(
                tab_hbm.at[idx_v.at[pl.ds(j * IDXW, IDXW)]],
                g_v.at[pl.ds(j * IDXW, IDXW), :],
                sem,
            )
            for j in range(NIDX)
        ]
        for cp in cps:
            cp.wait()
        pltpu.sync_copy(g_v, out_hbm.at[pl.ds(base, RPW), pl.ds(i * EMB, EMB)])
        return carry

    lax.fori_loop(0, N_EMB, col_body, 0)

    def pt_body(r, c):
        a = x_v[pl.ds(r * IN_DIM + N_EMB, L)].astype(jnp.float32)
        b = x_v[pl.ds(r * IN_DIM + N_EMB + 10, L)].astype(jnp.float32)
        f_v[r, pl.ds(0, L)] = a
        f_v[r, pl.ds(10, L)] = b
        return c
    lax.fori_loop(0, RPW, pt_body, 0)
    pltpu.sync_copy(f_v, out_hbm.at[pl.ds(base, RPW), pl.ds(N_EMB * EMB, N_EMB)])


_emb_kernel = functools.partial(
    pl.kernel,
    mesh=plsc.VectorSubcoreMesh(core_axis_name="c", subcore_axis_name="s"),
    out_type=jax.ShapeDtypeStruct((B, OUT_DIM), jnp.float32),
    compiler_params=pltpu.CompilerParams(use_tc_tiling_on_sc=False, needs_layout_passes=False),
    scratch_types=[
        pltpu.VMEM((RPW * IN_DIM,), jnp.int32),
        pltpu.VMEM((RPW,), jnp.int32),
        pltpu.VMEM((RPW, EMB), jnp.float32),
        pltpu.VMEM((RPW, N_EMB), jnp.float32),
        pltpu.SemaphoreType.DMA,
    ],
)(_body)


def kernel(x, emb_tables):
    tab = emb_tables.reshape(N_EMB * VOCAB, EMB)
    return _emb_kernel(x.reshape(B * IN_DIM), tab)
